# baseline (speedup 1.0000x reference)
import jax
import jax.numpy as jnp
from jax import lax
from jax.experimental import pallas as pl

_B, _S, _D, _C = 16, 80, 768, 3
_M = 32
_NOFF = _M * (_M - 1)

_TN = (((0,), (0,)), ((), ()))
_NT = (((1,), (1,)), ((), ()))
_HI = lax.Precision.HIGHEST


_BS = 4


def _i2f(v):
  return lax.bitcast_convert_type(v, jnp.float32)


def _tc_body(x_ref, hmr_ref, hmc_ref, sr_ref, srt_ref, w1_ref, w2t_ref,
             w3t_ref, bpair_ref, out_ref):
  for e in range(_BS):
    _tc_one(e, x_ref, hmr_ref, hmc_ref, sr_ref, srt_ref, w1_ref, w2t_ref,
            w3t_ref, bpair_ref, out_ref)


def _tc_one(e, x_ref, hmr_ref, hmc_ref, sr_ref, srt_ref, w1_ref, w2t_ref,
            w3t_ref, bpair_ref, out_ref):
  xb = x_ref[e]
  hm_row = hmr_ref[e]
  hm_col = hmc_ref[e]

  row_i = lax.broadcasted_iota(jnp.int32, (_S, _S), 0)
  col_j = lax.broadcasted_iota(jnp.int32, (_S, _S), 1)
  beats = ((hm_row > hm_col) |
           ((hm_row == hm_col) & (col_j < row_i))).astype(jnp.float32)
  rank_col = jnp.sum(beats, axis=1, keepdims=True)
  rank_row = (_S - 1) - jnp.sum(beats, axis=0, keepdims=True)

  r32_col = rank_col[:_M]
  r32_row = rank_row[:, :_M]
  pos_col = jnp.sum((r32_row < r32_col).astype(jnp.float32),
                    axis=1, keepdims=True)
  oh_pos = (pos_col == lax.broadcasted_iota(
      jnp.int32, (_M, _M), 1).astype(jnp.float32)).astype(jnp.float32)
  oh_rank = (r32_col == lax.broadcasted_iota(
      jnp.int32, (_M, _S), 1).astype(jnp.float32)).astype(jnp.float32)
  P = lax.dot_general(oh_pos, oh_rank, _TN)
  PT = lax.dot_general(oh_rank, oh_pos, _TN)

  x_rk = jnp.dot(P, xb)

  A = jnp.dot(x_rk, w1_ref[...])
  Bt = lax.dot_general(w2t_ref[...], x_rk, _NT)
  bp = bpair_ref[...]
  sig = []
  for c in range(_C):
    wc = w3t_ref[c:c + 1, :]
    Mc = lax.dot_general(x_rk * wc, x_rk, _NT)
    pre = Mc + A[:, c:c + 1] + Bt[c:c + 1, :] + bp[0, c]
    sig.append(jax.nn.sigmoid(pre))
  mx = jnp.maximum(jnp.maximum(sig[0], sig[1]), sig[2])
  es = [jnp.exp(s - mx) for s in sig]
  den = es[0] + es[1] + es[2]
  for c in range(_C):
    out_ref[e, c] = es[c] / den

  Pi = P.astype(jnp.int32)
  PTi = PT.astype(jnp.int32)
  sr0 = jnp.sum(Pi * srt_ref[0:1, :], axis=1, keepdims=True)
  sr1 = jnp.sum(Pi * srt_ref[1:2, :], axis=1, keepdims=True)
  sr0_row = jnp.sum(PTi * sr_ref[:, 0:1], axis=0, keepdims=True)
  sr1_row = jnp.sum(PTi * sr_ref[:, 1:2], axis=0, keepdims=True)
  zero_m = jnp.zeros((_M, _M), jnp.int32)
  out_ref[e, _C + 0] = _i2f(zero_m + sr0)
  out_ref[e, _C + 1] = _i2f(zero_m + sr1)
  out_ref[e, _C + 2] = _i2f(zero_m + sr0_row)
  out_ref[e, _C + 3] = _i2f(zero_m + sr1_row)


import functools


@functools.partial(jax.jit, static_argnames=("interpret",))
def _run(x, span_ranges, W_span, b_span, W_pair, b_pair, interpret=False):
  srT = span_ranges.T
  W1 = W_pair[:_D, :]
  W2T = W_pair[_D:2 * _D, :].T
  W3T = W_pair[2 * _D:, :].T
  bpair = b_pair.reshape(1, _C)
  hm = jax.nn.sigmoid(x @ W_span + b_span).mean(axis=-1)
  hmr = hm.reshape(_B, 1, _S)
  hmc = hm.reshape(_B, _S, 1)

  out_full = pl.pallas_call(
      _tc_body,
      grid=(_B // _BS,),
      in_specs=[
          pl.BlockSpec((_BS, _S, _D), lambda b: (b, 0, 0)),
          pl.BlockSpec((_BS, 1, _S), lambda b: (b, 0, 0)),
          pl.BlockSpec((_BS, _S, 1), lambda b: (b, 0, 0)),
          pl.BlockSpec((_S, 2), lambda b: (0, 0)),
          pl.BlockSpec((2, _S), lambda b: (0, 0)),
          pl.BlockSpec((_D, _C), lambda b: (0, 0)),
          pl.BlockSpec((_C, _D), lambda b: (0, 0)),
          pl.BlockSpec((_C, _D), lambda b: (0, 0)),
          pl.BlockSpec((1, _C), lambda b: (0, 0)),
      ],
      out_specs=pl.BlockSpec((_BS, _C + 4, _M, _M), lambda b: (b, 0, 0, 0)),
      out_shape=jax.ShapeDtypeStruct((_B, _C + 4, _M, _M), jnp.float32),
      interpret=interpret,
  )(x, hmr, hmc, span_ranges, srT, W1, W2T, W3T, bpair)

  full = out_full.reshape(_B, _C + 4, _M * _M).transpose(0, 2, 1)
  full = full[:, 1:, :].reshape(_B, _M - 1, _M + 1, _C + 4)[:, :, :_M, :]
  full = full.reshape(_B, _NOFF, _C + 4)
  logits = full[:, :, :_C]
  pr = lax.bitcast_convert_type(full[:, :, _C:], jnp.int32)
  pr = pr.reshape(_B, _NOFF, 2, 2)
  return logits, pr


def kernel(x, span_ranges, W_span, b_span, W_pair, b_pair):
  return _run(x, span_ranges, W_span, b_span, W_pair, b_pair)


# speedup vs baseline: 1.0307x; 1.0307x over previous
import jax
import jax.numpy as jnp
from jax import lax
from jax.experimental import pallas as pl

_B, _S, _D, _C = 16, 80, 768, 3
_M = 32
_NOFF = _M * (_M - 1)

_TN = (((0,), (0,)), ((), ()))
_NT = (((1,), (1,)), ((), ()))
_HI = lax.Precision.HIGHEST


_BS = 8


def _i2f(v):
  return lax.bitcast_convert_type(v, jnp.float32)


def _tc_body(x_ref, hmr_ref, hmc_ref, sr_ref, srt_ref, w1_ref, w2t_ref,
             w3t_ref, bpair_ref, out_ref):
  for e in range(_BS):
    _tc_one(e, x_ref, hmr_ref, hmc_ref, sr_ref, srt_ref, w1_ref, w2t_ref,
            w3t_ref, bpair_ref, out_ref)


def _tc_one(e, x_ref, hmr_ref, hmc_ref, sr_ref, srt_ref, w1_ref, w2t_ref,
            w3t_ref, bpair_ref, out_ref):
  xb = x_ref[e]
  hm_row = hmr_ref[e]
  hm_col = hmc_ref[e]

  row_i = lax.broadcasted_iota(jnp.int32, (_S, _S), 0)
  col_j = lax.broadcasted_iota(jnp.int32, (_S, _S), 1)
  beats = ((hm_row > hm_col) |
           ((hm_row == hm_col) & (col_j < row_i))).astype(jnp.float32)
  rank_col = jnp.sum(beats, axis=1, keepdims=True)
  rank_row = (_S - 1) - jnp.sum(beats, axis=0, keepdims=True)

  r32_col = rank_col[:_M]
  r32_row = rank_row[:, :_M]
  pos_col = jnp.sum((r32_row < r32_col).astype(jnp.float32),
                    axis=1, keepdims=True)
  oh_pos = (pos_col == lax.broadcasted_iota(
      jnp.int32, (_M, _M), 1).astype(jnp.float32)).astype(jnp.float32)
  oh_rank = (r32_col == lax.broadcasted_iota(
      jnp.int32, (_M, _S), 1).astype(jnp.float32)).astype(jnp.float32)
  P = lax.dot_general(oh_pos, oh_rank, _TN)
  PT = lax.dot_general(oh_rank, oh_pos, _TN)

  x_rk = jnp.dot(P, xb)

  A = jnp.dot(x_rk, w1_ref[...])
  Bt = lax.dot_general(w2t_ref[...], x_rk, _NT)
  bp = bpair_ref[...]
  sig = []
  for c in range(_C):
    wc = w3t_ref[c:c + 1, :]
    Mc = lax.dot_general(x_rk * wc, x_rk, _NT)
    pre = Mc + A[:, c:c + 1] + Bt[c:c + 1, :] + bp[0, c]
    sig.append(jax.nn.sigmoid(pre))
  mx = jnp.maximum(jnp.maximum(sig[0], sig[1]), sig[2])
  es = [jnp.exp(s - mx) for s in sig]
  den = es[0] + es[1] + es[2]
  for c in range(_C):
    out_ref[e, c] = es[c] / den

  Pi = P.astype(jnp.int32)
  PTi = PT.astype(jnp.int32)
  sr0 = jnp.sum(Pi * srt_ref[0:1, :], axis=1, keepdims=True)
  sr1 = jnp.sum(Pi * srt_ref[1:2, :], axis=1, keepdims=True)
  sr0_row = jnp.sum(PTi * sr_ref[:, 0:1], axis=0, keepdims=True)
  sr1_row = jnp.sum(PTi * sr_ref[:, 1:2], axis=0, keepdims=True)
  zero_m = jnp.zeros((_M, _M), jnp.int32)
  out_ref[e, _C + 0] = _i2f(zero_m + sr0)
  out_ref[e, _C + 1] = _i2f(zero_m + sr1)
  out_ref[e, _C + 2] = _i2f(zero_m + sr0_row)
  out_ref[e, _C + 3] = _i2f(zero_m + sr1_row)


import functools


@functools.partial(jax.jit, static_argnames=("interpret",))
def _run(x, span_ranges, W_span, b_span, W_pair, b_pair, interpret=False):
  srT = span_ranges.T
  W1 = W_pair[:_D, :]
  W2T = W_pair[_D:2 * _D, :].T
  W3T = W_pair[2 * _D:, :].T
  bpair = b_pair.reshape(1, _C)
  hm = jax.nn.sigmoid(x @ W_span + b_span).mean(axis=-1)
  hmr = hm.reshape(_B, 1, _S)
  hmc = hm.reshape(_B, _S, 1)

  out_full = pl.pallas_call(
      _tc_body,
      grid=(_B // _BS,),
      in_specs=[
          pl.BlockSpec((_BS, _S, _D), lambda b: (b, 0, 0)),
          pl.BlockSpec((_BS, 1, _S), lambda b: (b, 0, 0)),
          pl.BlockSpec((_BS, _S, 1), lambda b: (b, 0, 0)),
          pl.BlockSpec((_S, 2), lambda b: (0, 0)),
          pl.BlockSpec((2, _S), lambda b: (0, 0)),
          pl.BlockSpec((_D, _C), lambda b: (0, 0)),
          pl.BlockSpec((_C, _D), lambda b: (0, 0)),
          pl.BlockSpec((_C, _D), lambda b: (0, 0)),
          pl.BlockSpec((1, _C), lambda b: (0, 0)),
      ],
      out_specs=pl.BlockSpec((_BS, _C + 4, _M, _M), lambda b: (b, 0, 0, 0)),
      out_shape=jax.ShapeDtypeStruct((_B, _C + 4, _M, _M), jnp.float32),
      interpret=interpret,
  )(x, hmr, hmc, span_ranges, srT, W1, W2T, W3T, bpair)

  full = out_full.reshape(_B, _C + 4, _M * _M).transpose(0, 2, 1)
  full = full[:, 1:, :].reshape(_B, _M - 1, _M + 1, _C + 4)[:, :, :_M, :]
  full = full.reshape(_B, _NOFF, _C + 4)
  logits = full[:, :, :_C]
  pr = lax.bitcast_convert_type(full[:, :, _C:], jnp.int32)
  pr = pr.reshape(_B, _NOFF, 2, 2)
  return logits, pr


def kernel(x, span_ranges, W_span, b_span, W_pair, b_pair):
  return _run(x, span_ranges, W_span, b_span, W_pair, b_pair)
